# Initial kernel scaffold; baseline (speedup 1.0000x reference)
import functools

import jax
import jax.numpy as jnp
from jax.experimental import pallas as pl
from jax.experimental.pallas import tpu as pltpu

NUM_EXPERTS = 8
TOP_K = 2
D_MODEL = 1024
D_FF = 2048
F_BLK = 512
N_FB = D_FF // F_BLK


def _routing_weights(x, gate_w):
    logits = jax.lax.dot_general(
        x, gate_w, (((1,), (1,)), ((), ())),
        preferred_element_type=jnp.float32)
    t = logits.shape[0]
    e_iota = jax.lax.broadcasted_iota(jnp.int32, (t, NUM_EXPERTS), 1)
    m1 = jnp.max(logits, axis=1, keepdims=True)
    i1 = jnp.min(jnp.where(logits == m1, e_iota, NUM_EXPERTS), axis=1,
                 keepdims=True)
    l2 = jnp.where(e_iota == i1, -jnp.inf, logits)
    m2 = jnp.max(l2, axis=1, keepdims=True)
    i2 = jnp.min(jnp.where(l2 == m2, e_iota, NUM_EXPERTS), axis=1,
                 keepdims=True)
    sel = (e_iota == i1) | (e_iota == i2)
    p = jnp.exp(logits - m1)
    denom = 1.0 + jnp.exp(m2 - m1)
    return jnp.where(sel, p / denom, 0.0)


def _moe_kernel(x_ref, gate_ref, w1_ref, w3_ref, w2_ref, out_ref, dw_ref):
    e = pl.program_id(0)
    fb = pl.program_id(1)

    @pl.when((e == 0) & (fb == 0))
    def _init():
        dw_ref[...] = _routing_weights(x_ref[...], gate_ref[...])
        out_ref[...] = jnp.zeros_like(out_ref)

    x = x_ref[...]
    h1 = jax.lax.dot_general(x, w1_ref[0], (((1,), (1,)), ((), ())),
                             preferred_element_type=jnp.float32)
    h3 = jax.lax.dot_general(x, w3_ref[0], (((1,), (1,)), ((), ())),
                             preferred_element_type=jnp.float32)
    h = (h1 * jax.lax.logistic(h1)) * h3
    h = h * dw_ref[:, pl.ds(e, 1)]
    out_ref[...] += jax.lax.dot_general(
        h, w2_ref[0], (((1,), (1,)), ((), ())),
        preferred_element_type=jnp.float32)


def _moe(x, gate_w, w1, w3, w2):
    t = x.shape[0]
    return pl.pallas_call(
        _moe_kernel,
        grid=(NUM_EXPERTS, N_FB),
        in_specs=[
            pl.BlockSpec((t, D_MODEL), lambda e, fb: (0, 0)),
            pl.BlockSpec((NUM_EXPERTS, D_MODEL), lambda e, fb: (0, 0)),
            pl.BlockSpec((1, F_BLK, D_MODEL), lambda e, fb: (e, fb, 0)),
            pl.BlockSpec((1, F_BLK, D_MODEL), lambda e, fb: (e, fb, 0)),
            pl.BlockSpec((1, D_MODEL, F_BLK), lambda e, fb: (e, 0, fb)),
        ],
        out_specs=pl.BlockSpec((t, D_MODEL), lambda e, fb: (0, 0)),
        out_shape=jax.ShapeDtypeStruct((t, D_MODEL), jnp.float32),
        scratch_shapes=[pltpu.VMEM((t, NUM_EXPERTS), jnp.float32)],
        compiler_params=pltpu.CompilerParams(
            dimension_semantics=("arbitrary", "arbitrary")),
    )(x, gate_w, w1, w3, w2)


def kernel(hidden_states, gate_w, w1, w3, w2):
    b, s, d = hidden_states.shape
    x = hidden_states.reshape(-1, d)
    out = _moe(x, gate_w, w1, w3, w2)
    return out.reshape(b, s, d)


# speedup vs baseline: 1.5287x; 1.5287x over previous
import functools

import jax
import jax.numpy as jnp
from jax.experimental import pallas as pl
from jax.experimental.pallas import tpu as pltpu

NUM_EXPERTS = 8
TOP_K = 2
D_MODEL = 1024
D_FF = 2048
F_BLK = 512
N_FB = D_FF // F_BLK


def _routing_weights(x, gate_w):
    logits = jax.lax.dot_general(
        x, gate_w, (((1,), (1,)), ((), ())),
        preferred_element_type=jnp.float32)
    t = logits.shape[0]
    e_iota = jax.lax.broadcasted_iota(jnp.int32, (t, NUM_EXPERTS), 1)
    m1 = jnp.max(logits, axis=1, keepdims=True)
    i1 = jnp.min(jnp.where(logits == m1, e_iota, NUM_EXPERTS), axis=1,
                 keepdims=True)
    l2 = jnp.where(e_iota == i1, -jnp.inf, logits)
    m2 = jnp.max(l2, axis=1, keepdims=True)
    i2 = jnp.min(jnp.where(l2 == m2, e_iota, NUM_EXPERTS), axis=1,
                 keepdims=True)
    sel = (e_iota == i1) | (e_iota == i2)
    p = jnp.exp(logits - m1)
    denom = 1.0 + jnp.exp(m2 - m1)
    return jnp.where(sel, p / denom, 0.0)


def _moe_kernel(x_ref, gate_ref, w1_ref, w3_ref, w2_ref, out_ref, dw_ref):
    e = pl.program_id(0)
    fb = pl.program_id(1)

    @pl.when((e == 0) & (fb == 0))
    def _init():
        dw_ref[...] = _routing_weights(x_ref[...], gate_ref[...])
        out_ref[...] = jnp.zeros_like(out_ref)

    x = x_ref[...]
    h1 = jax.lax.dot_general(x, w1_ref[0], (((1,), (1,)), ((), ())),
                             preferred_element_type=jnp.float32)
    h3 = jax.lax.dot_general(x, w3_ref[0], (((1,), (1,)), ((), ())),
                             preferred_element_type=jnp.float32)
    h = (h1 * jax.lax.logistic(h1)) * h3
    dw = dw_ref[...]
    col = jax.lax.broadcasted_iota(jnp.int32, dw.shape, 1) == e
    h = h * jnp.sum(jnp.where(col, dw, 0.0), axis=1, keepdims=True)
    out_ref[...] += jax.lax.dot_general(
        h, w2_ref[0], (((1,), (1,)), ((), ())),
        preferred_element_type=jnp.float32)


def _moe(x, gate_w, w1, w3, w2):
    t = x.shape[0]
    return pl.pallas_call(
        _moe_kernel,
        grid=(NUM_EXPERTS, N_FB),
        in_specs=[
            pl.BlockSpec((t, D_MODEL), lambda e, fb: (0, 0)),
            pl.BlockSpec((NUM_EXPERTS, D_MODEL), lambda e, fb: (0, 0)),
            pl.BlockSpec((1, F_BLK, D_MODEL), lambda e, fb: (e, fb, 0)),
            pl.BlockSpec((1, F_BLK, D_MODEL), lambda e, fb: (e, fb, 0)),
            pl.BlockSpec((1, D_MODEL, F_BLK), lambda e, fb: (e, 0, fb)),
        ],
        out_specs=pl.BlockSpec((t, D_MODEL), lambda e, fb: (0, 0)),
        out_shape=jax.ShapeDtypeStruct((t, D_MODEL), jnp.float32),
        scratch_shapes=[pltpu.VMEM((t, NUM_EXPERTS), jnp.float32)],
        compiler_params=pltpu.CompilerParams(
            dimension_semantics=("arbitrary", "arbitrary")),
    )(x, gate_w, w1, w3, w2)


def kernel(hidden_states, gate_w, w1, w3, w2):
    b, s, d = hidden_states.shape
    x = hidden_states.reshape(-1, d)
    out = _moe(x, gate_w, w1, w3, w2)
    return out.reshape(b, s, d)
